# baseline (speedup 1.0000x reference)
import functools

import jax
import jax.numpy as jnp
from jax import lax
from jax.experimental import pallas as pl
from jax.experimental.pallas import tpu as pltpu
from jax.experimental.pallas import tpu_sc as plsc

_NC = 2
_NS = 16
_NW = _NC * _NS

_PD = 128
_ROWCHUNK = 16


@functools.partial(jax.jit, static_argnums=(2,))
def _gather_sc(idx, table_pad, d):
    b, f = idx.shape
    v, pd = table_pad.shape
    rows_w = b // _NW
    n_chunks = rows_w // _ROWCHUNK

    mesh = plsc.VectorSubcoreMesh(core_axis_name="c", subcore_axis_name="s")

    @functools.partial(
        pl.kernel,
        mesh=mesh,
        out_type=jax.ShapeDtypeStruct((b, f, d), jnp.float32),
        scratch_types=[
            pltpu.VMEM((rows_w, f), jnp.int32),
            pltpu.VMEM((2, _ROWCHUNK, f, pd), jnp.float32),
            pltpu.SemaphoreType.DMA,
            pltpu.SemaphoreType.DMA,
        ],
        compiler_params=pltpu.CompilerParams(use_tc_tiling_on_sc=False),
    )
    def body(idx_hbm, table_hbm, out_hbm, idx_v, rows_v, gsem, osem):
        wid = lax.axis_index("s") * _NC + lax.axis_index("c")
        base = wid * rows_w
        pltpu.sync_copy(idx_hbm.at[pl.ds(base, rows_w)], idx_v)

        def chunk_body(ci, carry):
            p = lax.rem(ci, 2)
            buf = rows_v.at[p]

            @pl.when(ci >= 2)
            def _():
                pltpu.make_async_copy(
                    rows_v.at[p, :, :, pl.ds(0, d)],
                    out_hbm.at[pl.ds(base + (ci - 2) * _ROWCHUNK, _ROWCHUNK)],
                    osem,
                ).wait()

            def fire(r, c):
                pltpu.async_copy(
                    table_hbm.at[idx_v.at[ci * _ROWCHUNK + r]],
                    buf.at[r],
                    gsem,
                )
                return c

            lax.fori_loop(0, _ROWCHUNK, fire, 0)

            def drain(r, c):
                pltpu.make_async_copy(
                    table_hbm.at[idx_v.at[0]], buf.at[0], gsem
                ).wait()
                return c

            lax.fori_loop(0, _ROWCHUNK, drain, 0)

            pltpu.make_async_copy(
                rows_v.at[p, :, :, pl.ds(0, d)],
                out_hbm.at[pl.ds(base + ci * _ROWCHUNK, _ROWCHUNK)],
                osem,
            ).start()
            return carry

        lax.fori_loop(0, n_chunks, chunk_body, 0)

        for ci in (n_chunks - 2, n_chunks - 1):
            pltpu.make_async_copy(
                rows_v.at[ci % 2, :, :, pl.ds(0, d)],
                out_hbm.at[pl.ds(base + ci * _ROWCHUNK, _ROWCHUNK)],
                osem,
            ).wait()

    return body(idx, table_pad)


def kernel(indices, table):
    v, d = table.shape
    table_pad = jnp.pad(table, ((0, 0), (0, _PD - d)))
    return _gather_sc(indices.astype(jnp.int32), table_pad, d)


# speedup vs baseline: 1.1365x; 1.1365x over previous
import functools

import jax
import jax.numpy as jnp
from jax import lax
from jax.experimental import pallas as pl
from jax.experimental.pallas import tpu as pltpu
from jax.experimental.pallas import tpu_sc as plsc

_NC = 2
_NS = 16
_NW = _NC * _NS

_ROWCHUNK = 64


@functools.partial(jax.jit, static_argnums=(2,))
def _gather_sc(idx, table_pad, d):
    b, f = idx.shape
    v, pd = table_pad.shape
    rows_w = b // _NW
    n_chunks = rows_w // _ROWCHUNK

    mesh = plsc.VectorSubcoreMesh(core_axis_name="c", subcore_axis_name="s")

    @functools.partial(
        pl.kernel,
        mesh=mesh,
        out_type=jax.ShapeDtypeStruct((b, f, d), jnp.float32),
        scratch_types=[
            pltpu.VMEM((rows_w, f), jnp.int32),
            pltpu.VMEM((2, _ROWCHUNK, f, pd), jnp.float32),
            pltpu.SemaphoreType.DMA,
            pltpu.SemaphoreType.DMA,
        ],
        compiler_params=pltpu.CompilerParams(use_tc_tiling_on_sc=False),
    )
    def body(idx_hbm, table_hbm, out_hbm, idx_v, rows_v, gsem, osem):
        wid = lax.axis_index("s") * _NC + lax.axis_index("c")
        base = wid * rows_w
        pltpu.sync_copy(idx_hbm.at[pl.ds(base, rows_w)], idx_v)

        def chunk_body(ci, carry):
            p = lax.rem(ci, 2)
            buf = rows_v.at[p]

            @pl.when(ci >= 2)
            def _():
                pltpu.make_async_copy(
                    rows_v.at[p, :, :, pl.ds(0, d)],
                    out_hbm.at[pl.ds(base + (ci - 2) * _ROWCHUNK, _ROWCHUNK)],
                    osem,
                ).wait()

            def fire(r, c):
                pltpu.async_copy(
                    table_hbm.at[idx_v.at[ci * _ROWCHUNK + r]],
                    buf.at[r],
                    gsem,
                )
                return c

            lax.fori_loop(0, _ROWCHUNK, fire, 0)

            def drain(r, c):
                pltpu.make_async_copy(
                    table_hbm.at[idx_v.at[0]], buf.at[0], gsem
                ).wait()
                return c

            lax.fori_loop(0, _ROWCHUNK, drain, 0)

            pltpu.make_async_copy(
                rows_v.at[p, :, :, pl.ds(0, d)],
                out_hbm.at[pl.ds(base + ci * _ROWCHUNK, _ROWCHUNK)],
                osem,
            ).start()
            return carry

        lax.fori_loop(0, n_chunks, chunk_body, 0)

        for ci in (n_chunks - 2, n_chunks - 1):
            pltpu.make_async_copy(
                rows_v.at[ci % 2, :, :, pl.ds(0, d)],
                out_hbm.at[pl.ds(base + ci * _ROWCHUNK, _ROWCHUNK)],
                osem,
            ).wait()

    return body(idx, table_pad)


def kernel(indices, table):
    v, d = table.shape
    flat = lax.optimization_barrier(table.reshape(v * d))
    table_lin = flat.reshape(v, d)
    return _gather_sc(indices.astype(jnp.int32), table_lin, d)


# speedup vs baseline: 1.1372x; 1.0007x over previous
import functools

import jax
import jax.numpy as jnp
from jax import lax
from jax.experimental import pallas as pl
from jax.experimental.pallas import tpu as pltpu
from jax.experimental.pallas import tpu_sc as plsc

_NC = 2
_NS = 16
_NW = _NC * _NS

_ROWCHUNK = 64


@functools.partial(jax.jit, static_argnums=(2,))
def _gather_sc(idx, table, d):
    b, f = idx.shape
    rows_w = b // _NW
    n_chunks = rows_w // _ROWCHUNK

    mesh = plsc.VectorSubcoreMesh(core_axis_name="c", subcore_axis_name="s")

    @functools.partial(
        pl.kernel,
        mesh=mesh,
        out_type=jax.ShapeDtypeStruct((b, f, d), jnp.float32),
        scratch_types=[
            pltpu.VMEM((rows_w, f), jnp.int32),
            pltpu.VMEM((2, _ROWCHUNK, f, d), jnp.float32),
            pltpu.SemaphoreType.DMA,
            pltpu.SemaphoreType.DMA,
        ],
        compiler_params=pltpu.CompilerParams(use_tc_tiling_on_sc=False),
    )
    def body(idx_hbm, table_hbm, out_hbm, idx_v, rows_v, gsem, osem):
        wid = lax.axis_index("s") * _NC + lax.axis_index("c")
        base = wid * rows_w
        pltpu.sync_copy(idx_hbm.at[pl.ds(base, rows_w)], idx_v)

        def chunk_body(ci, carry):
            p = lax.rem(ci, 2)
            buf = rows_v.at[p]

            @pl.when(ci >= 2)
            def _():
                pltpu.make_async_copy(
                    buf,
                    out_hbm.at[pl.ds(base + (ci - 2) * _ROWCHUNK, _ROWCHUNK)],
                    osem,
                ).wait()

            def fire(r, c):
                pltpu.async_copy(
                    table_hbm.at[idx_v.at[ci * _ROWCHUNK + r]],
                    buf.at[r],
                    gsem,
                )
                return c

            lax.fori_loop(0, _ROWCHUNK, fire, 0)

            def drain(r, c):
                pltpu.make_async_copy(
                    table_hbm.at[idx_v.at[0]], buf.at[0], gsem
                ).wait()
                return c

            lax.fori_loop(0, _ROWCHUNK, drain, 0)

            pltpu.make_async_copy(
                buf,
                out_hbm.at[pl.ds(base + ci * _ROWCHUNK, _ROWCHUNK)],
                osem,
            ).start()
            return carry

        lax.fori_loop(0, n_chunks, chunk_body, 0)

        for ci in (n_chunks - 2, n_chunks - 1):
            pltpu.make_async_copy(
                rows_v.at[ci % 2],
                out_hbm.at[pl.ds(base + ci * _ROWCHUNK, _ROWCHUNK)],
                osem,
            ).wait()

    return body(idx, table)


def kernel(indices, table):
    v, d = table.shape
    return _gather_sc(indices.astype(jnp.int32), table, d)
